# baseline (speedup 1.0000x reference)
import jax
import jax.numpy as jnp
from jax.experimental import pallas as pl
from jax.experimental.pallas import tpu as pltpu

_B, _S, _DIN, _R = 32, 4096, 8, 512
_NTAPS = 5
_MAXD = 168
_LEAK = 0.1
_T = 128
_NC = _S // _T
_BLK = 8
_NBLK = _T // _BLK


def _dot(a, b):
    return jax.lax.dot_general(a, b, (((1,), (0,)), ((), ())),
                               preferred_element_type=jnp.float32)


def _rowslice(ref, slot, nslots):
    idx = pl.multiple_of(slot * _B, _B)
    return ref.at[pl.ds(idx, nslots * _B), :]


def _reservoir_body(x_ref, wint_ref, wcat_ref, bias_ref, out_ref,
                    hist_ref, drive_ref, pre_a, pre_b):
    c = pl.program_id(0)

    @pl.when(c == 0)
    def _init():
        hist_ref[...] = jnp.zeros_like(hist_ref)

    drive_ref[...] = _dot(x_ref[...], wint_ref[...])

    w1 = wcat_ref[0 * _R:1 * _R, :]
    w4 = wcat_ref[1 * _R:2 * _R, :]
    w24 = wcat_ref[2 * _R:3 * _R, :]
    w96 = wcat_ref[3 * _R:4 * _R, :]
    w168 = wcat_ref[4 * _R:5 * _R, :]
    bias = bias_ref[...]
    base = c * _T

    def compute_pre(pre_ref, tg0, t0):
        a24 = _rowslice(hist_ref, jax.lax.rem(tg0 + _MAXD - 24, _MAXD), _BLK)
        a96 = _rowslice(hist_ref, jax.lax.rem(tg0 + _MAXD - 96, _MAXD), _BLK)
        a168 = _rowslice(hist_ref, jax.lax.rem(tg0 + _MAXD - 168, _MAXD),
                         _BLK)
        p8 = (_dot(a24[...], w24) + _dot(a96[...], w96)
              + _dot(a168[...], w168))
        pre_ref[...] = p8 + _rowslice(drive_ref, t0, _BLK)[...] + bias

    compute_pre(pre_a, base, 0)

    def blockpair(i, carry):
        h_prev, hb_prev = carry
        for cur, nxt in ((pre_a, pre_b), (pre_b, pre_a)):
            blk = 2 * i + (0 if cur is pre_a else 1)
            tg0 = base + _BLK * blk
            t0 = _BLK * blk

            compute_pre(nxt, tg0 + _BLK, jax.lax.rem(t0 + _BLK, _T))

            for sb in range(2):
                r4 = jax.lax.rem(tg0 + 4 * sb + _MAXD - 4, _MAXD)
                p4 = _dot(_rowslice(hist_ref, r4, 4)[...], w4)
                for s in range(4):
                    tg = tg0 + 4 * sb + s
                    fb = _dot(hb_prev, w1)
                    act = jnp.tanh(
                        cur[(4 * sb + s) * _B:(4 * sb + s + 1) * _B, :]
                        + p4[s * _B:(s + 1) * _B, :] + fb)
                    h_prev = (1.0 - _LEAK) * h_prev + _LEAK * act
                    hb_prev = h_prev.astype(jnp.bfloat16)
                    _rowslice(hist_ref, jax.lax.rem(tg, _MAXD), 1)[...] = \
                        hb_prev
                    _rowslice(out_ref, t0 + 4 * sb + s, 1)[...] = h_prev
        return h_prev, hb_prev

    hb0 = _rowslice(hist_ref, jax.lax.rem(base + _MAXD - 1, _MAXD), 1)[...]
    jax.lax.fori_loop(0, _NBLK // 2, blockpair,
                      (hb0.astype(jnp.float32), hb0))


def _run_reservoir(xt2, wint, wcat, bias2):
    return pl.pallas_call(
        _reservoir_body,
        out_shape=jax.ShapeDtypeStruct((_S * _B, _R), jnp.float32),
        grid=(_NC,),
        in_specs=[
            pl.BlockSpec((_T * _B, _DIN), lambda c: (c, 0)),
            pl.BlockSpec((_DIN, _R), lambda c: (0, 0)),
            pl.BlockSpec((_NTAPS * _R, _R), lambda c: (0, 0)),
            pl.BlockSpec((1, _R), lambda c: (0, 0)),
        ],
        out_specs=pl.BlockSpec((_T * _B, _R), lambda c: (c, 0)),
        scratch_shapes=[
            pltpu.VMEM((_MAXD * _B, _R), jnp.bfloat16),
            pltpu.VMEM((_T * _B, _R), jnp.float32),
            pltpu.VMEM((_BLK * _B, _R), jnp.float32),
            pltpu.VMEM((_BLK * _B, _R), jnp.float32),
        ],
        compiler_params=pltpu.CompilerParams(
            dimension_semantics=("arbitrary",),
            vmem_limit_bytes=56 * 1024 * 1024,
        ),
        name="delay_reservoir",
    )(xt2, wint, wcat, bias2)


def kernel(x, W_in, W_fb, bias):
    xt2 = jnp.reshape(jnp.swapaxes(x, 0, 1), (_S * _B, _DIN))
    wint = jnp.transpose(W_in).astype(jnp.bfloat16)
    wcat = jnp.reshape(W_fb, (_NTAPS * _R, _R)).astype(jnp.bfloat16)
    bias2 = jnp.reshape(bias, (1, _R))
    states = _run_reservoir(xt2, wint, wcat, bias2)
    return jnp.swapaxes(jnp.reshape(states, (_S, _B, _R)), 0, 1)


# speedup vs baseline: 1.1180x; 1.1180x over previous
import jax
import jax.numpy as jnp
from jax.experimental import pallas as pl
from jax.experimental.pallas import tpu as pltpu

_B, _S, _DIN, _R = 32, 4096, 8, 512
_NTAPS = 5
_MAXD = 168
_LEAK = 0.1
_T = 128
_NC = _S // _T
_BLK = 8
_NBLK = _T // _BLK


def _dot(a, b):
    return jax.lax.dot_general(a, b, (((1,), (0,)), ((), ())),
                               preferred_element_type=jnp.float32)


def _rowslice(ref, slot, nslots):
    idx = pl.multiple_of(slot * _B, _B)
    return ref.at[pl.ds(idx, nslots * _B), :]


def _reservoir_body(x_ref, wint_ref, wcat_ref, bias_ref, out_ref,
                    hist_ref, drive_ref, pre_a, pre_b, states_ref):
    c = pl.program_id(0)

    @pl.when(c == 0)
    def _init():
        hist_ref[...] = jnp.zeros_like(hist_ref)

    drive_ref[...] = _dot(x_ref[...], wint_ref[...])

    w1 = wcat_ref[0 * _R:1 * _R, :]
    w4 = wcat_ref[1 * _R:2 * _R, :]
    w24 = wcat_ref[2 * _R:3 * _R, :]
    w96 = wcat_ref[3 * _R:4 * _R, :]
    w168 = wcat_ref[4 * _R:5 * _R, :]
    bias = bias_ref[...]
    base = c * _T

    def compute_pre(pre_ref, tg0, t0):
        a24 = _rowslice(hist_ref, jax.lax.rem(tg0 + _MAXD - 24, _MAXD), _BLK)
        a96 = _rowslice(hist_ref, jax.lax.rem(tg0 + _MAXD - 96, _MAXD), _BLK)
        a168 = _rowslice(hist_ref, jax.lax.rem(tg0 + _MAXD - 168, _MAXD),
                         _BLK)
        p8 = (_dot(a24[...], w24) + _dot(a96[...], w96)
              + _dot(a168[...], w168))
        pre_ref[...] = p8 + _rowslice(drive_ref, t0, _BLK)[...] + bias

    compute_pre(pre_a, base, 0)

    def blockpair(i, carry):
        h_prev, hb_prev = carry
        for cur, nxt in ((pre_a, pre_b), (pre_b, pre_a)):
            blk = 2 * i + (0 if cur is pre_a else 1)
            tg0 = base + _BLK * blk
            t0 = _BLK * blk

            compute_pre(nxt, tg0 + _BLK, jax.lax.rem(t0 + _BLK, _T))

            for sb in range(2):
                r4 = jax.lax.rem(tg0 + 4 * sb + _MAXD - 4, _MAXD)
                p4 = _dot(_rowslice(hist_ref, r4, 4)[...], w4)
                for s in range(4):
                    tg = tg0 + 4 * sb + s
                    fb = _dot(hb_prev, w1)
                    act = jnp.tanh(
                        cur[(4 * sb + s) * _B:(4 * sb + s + 1) * _B, :]
                        + p4[s * _B:(s + 1) * _B, :] + fb)
                    h_prev = (1.0 - _LEAK) * h_prev + _LEAK * act
                    hb_prev = h_prev.astype(jnp.bfloat16)
                    _rowslice(hist_ref, jax.lax.rem(tg, _MAXD), 1)[...] = \
                        hb_prev
                    _rowslice(states_ref, t0 + 4 * sb + s, 1)[...] = h_prev
        return h_prev, hb_prev

    hb0 = _rowslice(hist_ref, jax.lax.rem(base + _MAXD - 1, _MAXD), 1)[...]
    jax.lax.fori_loop(0, _NBLK // 2, blockpair,
                      (hb0.astype(jnp.float32), hb0))

    out_ref[...] = jnp.swapaxes(
        states_ref[...].reshape(_T, _B, _R), 0, 1)


def _run_reservoir(xt2, wint, wcat, bias2):
    return pl.pallas_call(
        _reservoir_body,
        out_shape=jax.ShapeDtypeStruct((_B, _S, _R), jnp.float32),
        grid=(_NC,),
        in_specs=[
            pl.BlockSpec((_T * _B, _DIN), lambda c: (c, 0)),
            pl.BlockSpec((_DIN, _R), lambda c: (0, 0)),
            pl.BlockSpec((_NTAPS * _R, _R), lambda c: (0, 0)),
            pl.BlockSpec((1, _R), lambda c: (0, 0)),
        ],
        out_specs=pl.BlockSpec((_B, _T, _R), lambda c: (0, c, 0)),
        scratch_shapes=[
            pltpu.VMEM((_MAXD * _B, _R), jnp.bfloat16),
            pltpu.VMEM((_T * _B, _R), jnp.float32),
            pltpu.VMEM((_BLK * _B, _R), jnp.float32),
            pltpu.VMEM((_BLK * _B, _R), jnp.float32),
            pltpu.VMEM((_T * _B, _R), jnp.float32),
        ],
        compiler_params=pltpu.CompilerParams(
            dimension_semantics=("arbitrary",),
            vmem_limit_bytes=56 * 1024 * 1024,
        ),
        name="delay_reservoir",
    )(xt2, wint, wcat, bias2)


def kernel(x, W_in, W_fb, bias):
    xt2 = jnp.reshape(jnp.swapaxes(x, 0, 1), (_S * _B, _DIN))
    wint = jnp.transpose(W_in).astype(jnp.bfloat16)
    wcat = jnp.reshape(W_fb, (_NTAPS * _R, _R)).astype(jnp.bfloat16)
    bias2 = jnp.reshape(bias, (1, _R))
    return _run_reservoir(xt2, wint, wcat, bias2)


# speedup vs baseline: 1.1300x; 1.0107x over previous
import jax
import jax.numpy as jnp
from jax.experimental import pallas as pl
from jax.experimental.pallas import tpu as pltpu

_B, _S, _DIN, _R = 32, 4096, 8, 512
_NTAPS = 5
_MAXD = 168
_LEAK = 0.1
_T = 128
_NC = _S // _T
_BLK = 8
_NBLK = _T // _BLK


def _dot(a, b):
    return jax.lax.dot_general(a, b, (((1,), (0,)), ((), ())),
                               preferred_element_type=jnp.float32)


def _rowslice(ref, slot, nslots):
    idx = pl.multiple_of(slot * _B, _B)
    return ref.at[pl.ds(idx, nslots * _B), :]


def _reservoir_body(x_ref, wint_ref, wcat_ref, bias_ref, out_ref,
                    hist_ref, pre_a, pre_b, states_ref):
    c = pl.program_id(0)

    @pl.when(c == 0)
    def _init():
        hist_ref[...] = jnp.zeros_like(hist_ref)

    w1 = wcat_ref[0 * _R:1 * _R, :]
    w4 = wcat_ref[1 * _R:2 * _R, :]
    w24 = wcat_ref[2 * _R:3 * _R, :]
    w96 = wcat_ref[3 * _R:4 * _R, :]
    w168 = wcat_ref[4 * _R:5 * _R, :]
    bias = bias_ref[...]
    base = c * _T

    def compute_pre(pre_ref, tg0, t0):
        a24 = _rowslice(hist_ref, jax.lax.rem(tg0 + _MAXD - 24, _MAXD), _BLK)
        a96 = _rowslice(hist_ref, jax.lax.rem(tg0 + _MAXD - 96, _MAXD), _BLK)
        a168 = _rowslice(hist_ref, jax.lax.rem(tg0 + _MAXD - 168, _MAXD),
                         _BLK)
        p8 = (_dot(a24[...], w24) + _dot(a96[...], w96)
              + _dot(a168[...], w168)
              + _dot(_rowslice(x_ref, t0, _BLK)[...], wint_ref[...]))
        pre_ref[...] = p8 + bias

    compute_pre(pre_a, base, 0)

    def blockpair(i, h_prev):
        for cur, nxt in ((pre_a, pre_b), (pre_b, pre_a)):
            blk = 2 * i + (0 if cur is pre_a else 1)
            tg0 = base + _BLK * blk
            t0 = _BLK * blk

            compute_pre(nxt, tg0 + _BLK, jax.lax.rem(t0 + _BLK, _T))

            for sb in range(2):
                r4 = jax.lax.rem(tg0 + 4 * sb + _MAXD - 4, _MAXD)
                p4 = _dot(_rowslice(hist_ref, r4, 4)[...], w4)
                for s in range(4):
                    tg = tg0 + 4 * sb + s
                    fb = _dot(h_prev, w1)
                    act = jnp.tanh(
                        cur[(4 * sb + s) * _B:(4 * sb + s + 1) * _B, :]
                        + p4[s * _B:(s + 1) * _B, :] + fb)
                    h_prev = (1.0 - _LEAK) * h_prev + _LEAK * act
                    _rowslice(hist_ref, jax.lax.rem(tg, _MAXD), 1)[...] = \
                        h_prev.astype(jnp.bfloat16)
                    _rowslice(states_ref, t0 + 4 * sb + s, 1)[...] = h_prev
        return h_prev

    hb0 = _rowslice(hist_ref, jax.lax.rem(base + _MAXD - 1, _MAXD), 1)[...]
    jax.lax.fori_loop(0, _NBLK // 2, blockpair, hb0.astype(jnp.float32))

    out_ref[...] = jnp.swapaxes(
        states_ref[...].reshape(_T, _B, _R), 0, 1)


def _run_reservoir(xt2, wint, wcat, bias2):
    return pl.pallas_call(
        _reservoir_body,
        out_shape=jax.ShapeDtypeStruct((_B, _S, _R), jnp.float32),
        grid=(_NC,),
        in_specs=[
            pl.BlockSpec((_T * _B, _DIN), lambda c: (c, 0)),
            pl.BlockSpec((_DIN, _R), lambda c: (0, 0)),
            pl.BlockSpec((_NTAPS * _R, _R), lambda c: (0, 0)),
            pl.BlockSpec((1, _R), lambda c: (0, 0)),
        ],
        out_specs=pl.BlockSpec((_B, _T, _R), lambda c: (0, c, 0)),
        scratch_shapes=[
            pltpu.VMEM((_MAXD * _B, _R), jnp.bfloat16),
            pltpu.VMEM((_BLK * _B, _R), jnp.float32),
            pltpu.VMEM((_BLK * _B, _R), jnp.float32),
            pltpu.VMEM((_T * _B, _R), jnp.float32),
        ],
        compiler_params=pltpu.CompilerParams(
            dimension_semantics=("arbitrary",),
            vmem_limit_bytes=56 * 1024 * 1024,
        ),
        name="delay_reservoir",
    )(xt2, wint, wcat, bias2)


def kernel(x, W_in, W_fb, bias):
    xt2 = jnp.reshape(jnp.swapaxes(x, 0, 1),
                      (_S * _B, _DIN)).astype(jnp.bfloat16)
    wint = jnp.transpose(W_in).astype(jnp.bfloat16)
    wcat = jnp.reshape(W_fb, (_NTAPS * _R, _R)).astype(jnp.bfloat16)
    bias2 = jnp.reshape(bias, (1, _R))
    return _run_reservoir(xt2, wint, wcat, bias2)


# speedup vs baseline: 1.1405x; 1.0093x over previous
import jax
import jax.numpy as jnp
from jax.experimental import pallas as pl
from jax.experimental.pallas import tpu as pltpu

_B, _S, _DIN, _R = 32, 4096, 8, 512
_NTAPS = 5
_MAXD = 168
_LEAK = 0.1
_T = 128
_NC = _S // _T
_BLK = 8
_NBLK = _T // _BLK


def _dot(a, b):
    return jax.lax.dot_general(a, b, (((1,), (0,)), ((), ())),
                               preferred_element_type=jnp.float32)


def _rowslice(ref, slot, nslots):
    idx = pl.multiple_of(slot * _B, _B)
    return ref.at[pl.ds(idx, nslots * _B), :]


def _reservoir_body(x_ref, wint_ref, wcat_ref, bias_ref, out_ref,
                    hist_ref, pre_a, pre_b, states_ref):
    c = pl.program_id(0)

    @pl.when(c == 0)
    def _init():
        hist_ref[...] = jnp.zeros_like(hist_ref)

    w1 = wcat_ref[0 * _R:1 * _R, :]
    w4 = wcat_ref[1 * _R:2 * _R, :]
    w24 = wcat_ref[2 * _R:3 * _R, :]
    w96 = wcat_ref[3 * _R:4 * _R, :]
    w168 = wcat_ref[4 * _R:5 * _R, :]
    bias = bias_ref[...]
    base = c * _T

    def compute_pre(pre_ref, tg0, t0):
        a24 = _rowslice(hist_ref, jax.lax.rem(tg0 + _MAXD - 24, _MAXD), _BLK)
        a96 = _rowslice(hist_ref, jax.lax.rem(tg0 + _MAXD - 96, _MAXD), _BLK)
        a168 = _rowslice(hist_ref, jax.lax.rem(tg0 + _MAXD - 168, _MAXD),
                         _BLK)
        p8 = (_dot(a24[...], w24) + _dot(a96[...], w96)
              + _dot(a168[...], w168)
              + _dot(_rowslice(x_ref, t0, _BLK)[...], wint_ref[...]))
        pre_ref[...] = p8 + bias

    compute_pre(pre_a, base, 0)

    def blockpair(i, h_prev):
        for cur, nxt in ((pre_a, pre_b), (pre_b, pre_a)):
            blk = 2 * i + (0 if cur is pre_a else 1)
            tg0 = base + _BLK * blk
            t0 = _BLK * blk

            compute_pre(nxt, tg0 + _BLK, jax.lax.rem(t0 + _BLK, _T))

            for sb in range(2):
                r4 = jax.lax.rem(tg0 + 4 * sb + _MAXD - 4, _MAXD)
                p4 = _dot(_rowslice(hist_ref, r4, 4)[...], w4)
                for s in range(4):
                    tg = tg0 + 4 * sb + s
                    fb = _dot(h_prev, w1)
                    act = jnp.tanh(
                        cur[(4 * sb + s) * _B:(4 * sb + s + 1) * _B, :]
                        + p4[s * _B:(s + 1) * _B, :] + fb)
                    h_prev = (1.0 - _LEAK) * h_prev + _LEAK * act
                    _rowslice(hist_ref, jax.lax.rem(tg, _MAXD), 1)[...] = \
                        h_prev.astype(jnp.bfloat16)
                    _rowslice(states_ref, t0 + 4 * sb + s, 1)[...] = h_prev

        tp0 = pl.multiple_of(2 * _BLK * i, 2 * _BLK)
        st = states_ref[pl.ds(tp0 * _B, 2 * _BLK * _B), :]
        out_ref[:, pl.ds(tp0, 2 * _BLK), :] = jnp.swapaxes(
            st.reshape(2 * _BLK, _B, _R), 0, 1)
        return h_prev

    hb0 = _rowslice(hist_ref, jax.lax.rem(base + _MAXD - 1, _MAXD), 1)[...]
    jax.lax.fori_loop(0, _NBLK // 2, blockpair, hb0.astype(jnp.float32))


def _run_reservoir(xt2, wint, wcat, bias2):
    return pl.pallas_call(
        _reservoir_body,
        out_shape=jax.ShapeDtypeStruct((_B, _S, _R), jnp.float32),
        grid=(_NC,),
        in_specs=[
            pl.BlockSpec((_T * _B, _DIN), lambda c: (c, 0)),
            pl.BlockSpec((_DIN, _R), lambda c: (0, 0)),
            pl.BlockSpec((_NTAPS * _R, _R), lambda c: (0, 0)),
            pl.BlockSpec((1, _R), lambda c: (0, 0)),
        ],
        out_specs=pl.BlockSpec((_B, _T, _R), lambda c: (0, c, 0)),
        scratch_shapes=[
            pltpu.VMEM((_MAXD * _B, _R), jnp.bfloat16),
            pltpu.VMEM((_BLK * _B, _R), jnp.float32),
            pltpu.VMEM((_BLK * _B, _R), jnp.float32),
            pltpu.VMEM((_T * _B, _R), jnp.float32),
        ],
        compiler_params=pltpu.CompilerParams(
            dimension_semantics=("arbitrary",),
            vmem_limit_bytes=56 * 1024 * 1024,
        ),
        name="delay_reservoir",
    )(xt2, wint, wcat, bias2)


def kernel(x, W_in, W_fb, bias):
    xt2 = jnp.reshape(jnp.swapaxes(x, 0, 1),
                      (_S * _B, _DIN)).astype(jnp.bfloat16)
    wint = jnp.transpose(W_in).astype(jnp.bfloat16)
    wcat = jnp.reshape(W_fb, (_NTAPS * _R, _R)).astype(jnp.bfloat16)
    bias2 = jnp.reshape(bias, (1, _R))
    return _run_reservoir(xt2, wint, wcat, bias2)
